# baseline (speedup 1.0000x reference)
import functools

import jax
import jax.numpy as jnp
from jax import lax
from jax.experimental import pallas as pl
from jax.experimental.pallas import tpu as pltpu
from jax.experimental.pallas import tpu_sc as plsc

_LANES = 16
_CHUNK = 128


def _make_loss_kernel(num_workers, nc, b_per_w, k_dim):
    n_chunks = b_per_w // _CHUNK
    n_groups = b_per_w // _LANES
    mesh = plsc.VectorSubcoreMesh(core_axis_name="c", subcore_axis_name="s")

    @functools.partial(
        pl.kernel,
        mesh=mesh,
        out_type=jax.ShapeDtypeStruct((num_workers, _LANES), jnp.float32),
        compiler_params=pltpu.CompilerParams(
            needs_layout_passes=False, use_tc_tiling_on_sc=False),
        scratch_types=[
            pltpu.VMEM((n_chunks, _CHUNK), jnp.int32),
            pltpu.VMEM((n_chunks, _CHUNK), jnp.int32),
            pltpu.VMEM((b_per_w,), jnp.float32),
            pltpu.VMEM((_LANES,), jnp.float32),
            pltpu.VMEM((b_per_w,), jnp.float32),
            pltpu.VMEM((b_per_w,), jnp.float32),
            pltpu.VMEM((b_per_w, k_dim), jnp.float32),
            pltpu.VMEM((b_per_w, k_dim), jnp.float32),
            pltpu.VMEM((_LANES,), jnp.float32),
            pltpu.SemaphoreType.DMA,
        ],
    )
    def loss_kernel(su_hbm, si_hbm, r_hbm, av_hbm, bU_hbm, bI_hbm, gU_hbm,
                    gI_hbm, out_hbm, idx_u, idx_i, r_v, a_v, bu_v, bi_v,
                    gu_v, gi_v, loss_v, sem):
        wid = lax.axis_index("s") * nc + lax.axis_index("c")
        pltpu.sync_copy(su_hbm.at[wid], idx_u)
        pltpu.sync_copy(si_hbm.at[wid], idx_i)
        pltpu.sync_copy(r_hbm.at[wid], r_v)
        pltpu.sync_copy(av_hbm, a_v)
        copies = []
        for j in range(n_chunks):
            sl = pl.ds(j * _CHUNK, _CHUNK)
            copies.append(pltpu.async_copy(gU_hbm.at[idx_u.at[j]], gu_v.at[sl], sem))
            copies.append(pltpu.async_copy(gI_hbm.at[idx_i.at[j]], gi_v.at[sl], sem))
            copies.append(pltpu.async_copy(bU_hbm.at[idx_u.at[j]], bu_v.at[sl], sem))
            copies.append(pltpu.async_copy(bI_hbm.at[idx_i.at[j]], bi_v.at[sl], sem))
        for c in copies:
            c.wait()
        alpha = a_v[...]

        def group(g, acc_loss):
            base = g * _LANES
            rows = base + lax.iota(jnp.int32, _LANES)
            dot = jnp.zeros((_LANES,), jnp.float32)
            for k in range(k_dim):
                cols = jnp.full((_LANES,), k, jnp.int32)
                u_k = plsc.load_gather(gu_v, [rows, cols])
                i_k = plsc.load_gather(gi_v, [rows, cols])
                dot = dot + u_k * i_k
            sl = pl.ds(base, _LANES)
            pred = alpha + bu_v[sl] + bi_v[sl] + dot
            sig = 5.0 / (1.0 + jnp.exp(-pred))
            diff = sig - r_v[sl]
            return acc_loss + diff * diff

        acc = lax.fori_loop(0, n_groups, group,
                            jnp.zeros((_LANES,), jnp.float32))
        loss_v[...] = acc
        pltpu.sync_copy(loss_v, out_hbm.at[wid])

    return loss_kernel


def kernel(sampleU, sampleI, sampleR, alpha, betaU, betaI, gammaU, gammaI):
    info = plsc.get_sparse_core_info()
    nc, ns = info.num_cores, info.num_subcores
    nw = nc * ns
    b = sampleU.shape[0]
    k_dim = gammaU.shape[1]
    b_per_w = b // nw
    su = sampleU.astype(jnp.int32).reshape(nw, b_per_w // _CHUNK, _CHUNK)
    si = sampleI.astype(jnp.int32).reshape(nw, b_per_w // _CHUNK, _CHUNK)
    r = sampleR.astype(jnp.float32).reshape(nw, b_per_w)
    av = jnp.broadcast_to(jnp.asarray(alpha, jnp.float32), (_LANES,))
    fn = _make_loss_kernel(nw, nc, b_per_w, k_dim)
    out = fn(su, si, r, av, betaU, betaI, gammaU, gammaI)
    return jnp.sum(out) / b
